# baseline (speedup 1.0000x reference)
import jax
import jax.numpy as jnp
from jax import lax
from jax.experimental import pallas as pl
from jax.experimental.pallas import tpu as pltpu
from jax.experimental.pallas import tpu_sc as plsc

B = 512
N = 45000
E = 100000
NC = 2
NS = 16
NW = NC * NS
PK = B // 2
PK_PER_W = PK // NW
N_PASS = PK_PER_W // 2
CHUNK = 4000
N_CHUNKS = E // CHUNK
VECS = CHUNK // 16


def _sc_kernel(packed_hbm, child_hbm, parent_hbm, out_hbm,
               pkA_v, pkB_v, ci0_v, pi0_v, ci1_v, pi1_v, out_v,
               sem0, sem1, semr):
    wid = lax.axis_index("s") * NC + lax.axis_index("c")
    pk_base = wid * PK_PER_W

    def start_idx(ci_v, pi_v, sem, ch):
        off = ch * CHUNK
        pltpu.make_async_copy(child_hbm.at[pl.ds(off, CHUNK)], ci_v, sem).start()
        pltpu.make_async_copy(parent_hbm.at[pl.ds(off, CHUNK)], pi_v, sem).start()

    def wait_idx(ci_v, pi_v, sem):
        pltpu.make_async_copy(child_hbm.at[pl.ds(0, CHUNK)], ci_v, sem).wait()
        pltpu.make_async_copy(parent_hbm.at[pl.ds(0, CHUNK)], pi_v, sem).wait()

    def chunk_compute(ci_v, pi_v, accs):
        def vec_body(i, accs):
            acc0, acc1 = accs
            ci = ci_v[pl.ds(i * 16, 16)]
            pi = pi_v[pl.ds(i * 16, 16)]
            cwA = plsc.load_gather(pkA_v, [ci])
            pwA = plsc.load_gather(pkA_v, [pi])
            cwB = plsc.load_gather(pkB_v, [ci])
            pwB = plsc.load_gather(pkB_v, [pi])
            zero = jnp.zeros((16,), jnp.float32)
            dA_hi = plsc.bitcast(cwA, jnp.float32) - plsc.bitcast(pwA, jnp.float32)
            dA_lo = plsc.bitcast(cwA << 16, jnp.float32) - plsc.bitcast(pwA << 16, jnp.float32)
            dB_hi = plsc.bitcast(cwB, jnp.float32) - plsc.bitcast(pwB, jnp.float32)
            dB_lo = plsc.bitcast(cwB << 16, jnp.float32) - plsc.bitcast(pwB << 16, jnp.float32)
            acc0 = acc0 + jnp.maximum(dA_hi, zero) + jnp.maximum(dB_hi, zero)
            acc1 = acc1 + jnp.maximum(dA_lo, zero) + jnp.maximum(dB_lo, zero)
            return acc0, acc1

        return lax.fori_loop(0, VECS, vec_body, accs)

    zero = jnp.zeros((16,), jnp.float32)
    accs = (zero, zero)
    for pp in range(N_PASS):
        r0 = pk_base + 2 * pp
        pltpu.make_async_copy(packed_hbm.at[pl.ds(r0 * N, N)], pkA_v, semr).start()
        pltpu.make_async_copy(packed_hbm.at[pl.ds((r0 + 1) * N, N)], pkB_v, semr).start()
        pltpu.make_async_copy(packed_hbm.at[pl.ds(0, N)], pkA_v, semr).wait()
        pltpu.make_async_copy(packed_hbm.at[pl.ds(0, N)], pkB_v, semr).wait()

        start_idx(ci0_v, pi0_v, sem0, 0)

        def pair_body(j, accs):
            start_idx(ci1_v, pi1_v, sem1, 2 * j + 1)
            wait_idx(ci0_v, pi0_v, sem0)
            accs = chunk_compute(ci0_v, pi0_v, accs)
            start_idx(ci0_v, pi0_v, sem0, 2 * j + 2)
            wait_idx(ci1_v, pi1_v, sem1)
            accs = chunk_compute(ci1_v, pi1_v, accs)
            return accs

        accs = lax.fori_loop(0, (N_CHUNKS - 1) // 2, pair_body, accs)
        wait_idx(ci0_v, pi0_v, sem0)
        accs = chunk_compute(ci0_v, pi0_v, accs)

    acc0, acc1 = accs
    out_v[...] = acc0 + acc1
    pltpu.sync_copy(out_v, out_hbm.at[wid])


@jax.jit
def _hierarchical_loss(probs, child, parent):
    bits = jax.lax.bitcast_convert_type(probs, jnp.uint32)
    packed = jax.lax.bitcast_convert_type(
        (bits[PK:] & jnp.uint32(0xFFFF0000)) | (bits[:PK] >> 16),
        jnp.int32).reshape(PK * N)

    mesh = plsc.VectorSubcoreMesh(core_axis_name="c", subcore_axis_name="s",
                                  num_cores=NC, num_subcores=NS)
    partials = pl.kernel(
        _sc_kernel,
        out_type=jax.ShapeDtypeStruct((NW, 16), jnp.float32),
        mesh=mesh,
        compiler_params=pltpu.CompilerParams(needs_layout_passes=False),
        scratch_types=[
            pltpu.VMEM((N,), jnp.int32),
            pltpu.VMEM((N,), jnp.int32),
            pltpu.VMEM((CHUNK,), jnp.int32),
            pltpu.VMEM((CHUNK,), jnp.int32),
            pltpu.VMEM((CHUNK,), jnp.int32),
            pltpu.VMEM((CHUNK,), jnp.int32),
            pltpu.VMEM((16,), jnp.float32),
            pltpu.SemaphoreType.DMA,
            pltpu.SemaphoreType.DMA,
            pltpu.SemaphoreType.DMA,
        ],
    )(packed, child, parent)
    return jnp.sum(partials)


def kernel(probs, edge_index):
    child = edge_index[0].astype(jnp.int32)
    parent = edge_index[1].astype(jnp.int32)
    return _hierarchical_loss(probs, child, parent)


# speedup vs baseline: 2.2235x; 2.2235x over previous
import jax
import jax.numpy as jnp
from jax import lax
from jax.experimental import pallas as pl
from jax.experimental.pallas import tpu as pltpu
from jax.experimental.pallas import tpu_sc as plsc

B = 512
N = 45000
E = 100000
NC = 2
NS = 16
NW = NC * NS
PK = B // 2
PK_PER_W = PK // NW
N_PASS = PK_PER_W // 2
CHUNK = 4000
N_CHUNKS = E // CHUNK
VECS = CHUNK // 16


def _sc_kernel(packed_hbm, child_hbm, parent_hbm, out_hbm,
               pkA_v, pkB_v, ci0_v, pi0_v, ci1_v, pi1_v, out_v,
               sem0, sem1, semr):
    wid = lax.axis_index("s") * NC + lax.axis_index("c")
    pk_base = wid * PK_PER_W

    def start_idx(ci_v, pi_v, sem, ch):
        off = ch * CHUNK
        pltpu.make_async_copy(child_hbm.at[pl.ds(off, CHUNK)], ci_v, sem).start()
        pltpu.make_async_copy(parent_hbm.at[pl.ds(off, CHUNK)], pi_v, sem).start()

    def wait_idx(ci_v, pi_v, sem):
        pltpu.make_async_copy(child_hbm.at[pl.ds(0, CHUNK)], ci_v, sem).wait()
        pltpu.make_async_copy(parent_hbm.at[pl.ds(0, CHUNK)], pi_v, sem).wait()

    def chunk_compute(ci_v, pi_v, accs):
        def vec_body(i, accs):
            acc0, acc1 = accs
            ci = ci_v[pl.ds(i * 16, 16)]
            pi = pi_v[pl.ds(i * 16, 16)]
            cwA = plsc.load_gather(pkA_v, [ci])
            pwA = plsc.load_gather(pkA_v, [pi])
            cwB = plsc.load_gather(pkB_v, [ci])
            pwB = plsc.load_gather(pkB_v, [pi])
            zero = jnp.zeros((16,), jnp.float32)
            dA_hi = plsc.bitcast(cwA, jnp.float32) - plsc.bitcast(pwA, jnp.float32)
            dA_lo = plsc.bitcast(cwA << 16, jnp.float32) - plsc.bitcast(pwA << 16, jnp.float32)
            dB_hi = plsc.bitcast(cwB, jnp.float32) - plsc.bitcast(pwB, jnp.float32)
            dB_lo = plsc.bitcast(cwB << 16, jnp.float32) - plsc.bitcast(pwB << 16, jnp.float32)
            acc0 = acc0 + jnp.maximum(dA_hi, zero) + jnp.maximum(dB_hi, zero)
            acc1 = acc1 + jnp.maximum(dA_lo, zero) + jnp.maximum(dB_lo, zero)
            return acc0, acc1

        return lax.fori_loop(0, VECS, vec_body, accs)

    zero = jnp.zeros((16,), jnp.float32)
    accs = (zero, zero)
    for pp in range(N_PASS):
        r0 = pk_base + 2 * pp
        pltpu.make_async_copy(packed_hbm.at[r0], pkA_v, semr).start()
        pltpu.make_async_copy(packed_hbm.at[r0 + 1], pkB_v, semr).start()
        pltpu.make_async_copy(packed_hbm.at[0], pkA_v, semr).wait()
        pltpu.make_async_copy(packed_hbm.at[0], pkB_v, semr).wait()

        start_idx(ci0_v, pi0_v, sem0, 0)

        def pair_body(j, accs):
            start_idx(ci1_v, pi1_v, sem1, 2 * j + 1)
            wait_idx(ci0_v, pi0_v, sem0)
            accs = chunk_compute(ci0_v, pi0_v, accs)
            start_idx(ci0_v, pi0_v, sem0, 2 * j + 2)
            wait_idx(ci1_v, pi1_v, sem1)
            accs = chunk_compute(ci1_v, pi1_v, accs)
            return accs

        accs = lax.fori_loop(0, (N_CHUNKS - 1) // 2, pair_body, accs)
        wait_idx(ci0_v, pi0_v, sem0)
        accs = chunk_compute(ci0_v, pi0_v, accs)

    acc0, acc1 = accs
    out_v[...] = acc0 + acc1
    pltpu.sync_copy(out_v, out_hbm.at[wid])


@jax.jit
def _hierarchical_loss(probs, child, parent):
    bits = jax.lax.bitcast_convert_type(probs, jnp.uint32)
    packed = jax.lax.bitcast_convert_type(
        (bits[PK:] & jnp.uint32(0xFFFF0000)) | (bits[:PK] >> 16), jnp.int32)

    mesh = plsc.VectorSubcoreMesh(core_axis_name="c", subcore_axis_name="s",
                                  num_cores=NC, num_subcores=NS)
    partials = pl.kernel(
        _sc_kernel,
        out_type=jax.ShapeDtypeStruct((NW, 16), jnp.float32),
        mesh=mesh,
        compiler_params=pltpu.CompilerParams(needs_layout_passes=False),
        scratch_types=[
            pltpu.VMEM((N,), jnp.int32),
            pltpu.VMEM((N,), jnp.int32),
            pltpu.VMEM((CHUNK,), jnp.int32),
            pltpu.VMEM((CHUNK,), jnp.int32),
            pltpu.VMEM((CHUNK,), jnp.int32),
            pltpu.VMEM((CHUNK,), jnp.int32),
            pltpu.VMEM((16,), jnp.float32),
            pltpu.SemaphoreType.DMA,
            pltpu.SemaphoreType.DMA,
            pltpu.SemaphoreType.DMA,
        ],
    )(packed, child, parent)
    return jnp.sum(partials)


def kernel(probs, edge_index):
    child = edge_index[0].astype(jnp.int32)
    parent = edge_index[1].astype(jnp.int32)
    return _hierarchical_loss(probs, child, parent)
